# Initial kernel scaffold; baseline (speedup 1.0000x reference)
import jax
import jax.numpy as jnp
from jax.experimental import pallas as pl


def kernel(flow, coords, grid_coords):
    raise NotImplementedError("write your pallas kernel here")


# speedup vs baseline: 13.0177x; 13.0177x over previous
import jax
import jax.numpy as jnp
import numpy as np
from jax.experimental import pallas as pl

_NB = 8
_SP = 24
_N = 2048
_TM = 256


def _knn_tile_kernel(gc_ref, pct_ref, val_ref, out_ref):
    gc = gc_ref[0]
    pct = pct_ref[0]
    val = val_ref[0]
    gn = jnp.sum(gc * gc, axis=1, keepdims=True)
    pn = jnp.sum(pct * pct, axis=0, keepdims=True)
    dot = jax.lax.dot_general(
        gc, pct, (((1,), (0,)), ((), ())),
        precision=jax.lax.Precision.DEFAULT,
        preferred_element_type=jnp.float32)
    d2 = gn + pn - 2.0 * dot
    col = jax.lax.broadcasted_iota(jnp.int32, d2.shape, 1)
    work = d2
    wsel = jnp.zeros_like(d2)
    for _ in range(_NB):
        m = jnp.min(work, axis=1, keepdims=True)
        eq = work == m
        cand = jnp.where(eq, col, _N)
        amin = jnp.min(cand, axis=1, keepdims=True)
        sel = col == amin
        wi = 1.0 / jnp.square(jnp.sqrt(jnp.maximum(m, 0.0)) + 1e-8)
        wsel = jnp.where(sel, wi, wsel)
        work = jnp.where(sel, jnp.float32(jnp.inf), work)
    denom = jnp.sum(wsel, axis=1, keepdims=True)
    wn = wsel / denom
    out = jax.lax.dot_general(
        wn, val, (((1,), (0,)), ((), ())),
        precision=jax.lax.Precision.HIGHEST,
        preferred_element_type=jnp.float32)
    out_ref[0] = out


def _div_kernel(fx_ref, fy_ref, fz_ref, out_ref):
    h = 2.0 * np.pi / _SP
    fx = fx_ref[...]
    fy = fy_ref[...]
    fz = fz_ref[...]
    row = jax.lax.broadcasted_iota(jnp.int32, fx.shape, 0)
    y = row % _SP
    x = (row // _SP) % _SP
    upx = jnp.concatenate([fx[_SP:], fx[-_SP:]], axis=0)
    dnx = jnp.concatenate([fx[:_SP], fx[:-_SP]], axis=0)
    gx = (upx - dnx) / (2.0 * h)
    gx = jnp.where(x == 0, (upx - fx) / h, gx)
    gx = jnp.where(x == _SP - 1, (fx - dnx) / h, gx)
    upy = jnp.concatenate([fy[1:], fy[-1:]], axis=0)
    dny = jnp.concatenate([fy[:1], fy[:-1]], axis=0)
    gy = (upy - dny) / (2.0 * h)
    gy = jnp.where(y == 0, (upy - fy) / h, gy)
    gy = jnp.where(y == _SP - 1, (fy - dny) / h, gy)
    zc = (fz[:, 2:] - fz[:, :-2]) / (2.0 * h)
    z0 = (fz[:, 1:2] - fz[:, 0:1]) / h
    z1 = (fz[:, -1:] - fz[:, -2:-1]) / h
    gz = jnp.concatenate([z0, zc, z1], axis=1)
    div = gx + gy + gz
    out_ref[...] = jnp.broadcast_to(jnp.mean(jnp.abs(div)), (1, 1))


def kernel(flow, coords, grid_coords):
    B, N, _ = coords.shape
    M = grid_coords.shape[1]
    coords_t = jnp.transpose(coords, (0, 2, 1))
    nt = M // _TM
    gf = pl.pallas_call(
        _knn_tile_kernel,
        grid=(B, nt),
        in_specs=[
            pl.BlockSpec((1, _TM, 3), lambda b, i: (b, i, 0)),
            pl.BlockSpec((1, 3, N), lambda b, i: (b, 0, 0)),
            pl.BlockSpec((1, N, 3), lambda b, i: (b, 0, 0)),
        ],
        out_specs=pl.BlockSpec((1, _TM, 3), lambda b, i: (b, i, 0)),
        out_shape=jax.ShapeDtypeStruct((B, M, 3), jnp.float32),
    )(grid_coords, coords_t, flow)
    g = gf.reshape(B, _SP, _SP, _SP, 3)
    fx = g[..., 0].reshape(B * _SP * _SP, _SP)
    fy = g[..., 1].reshape(B * _SP * _SP, _SP)
    fz = g[..., 2].reshape(B * _SP * _SP, _SP)
    out = pl.pallas_call(
        _div_kernel,
        out_shape=jax.ShapeDtypeStruct((1, 1), jnp.float32),
    )(fx, fy, fz)
    return out[0, 0]


# speedup vs baseline: 19.0592x; 1.4641x over previous
import jax
import jax.numpy as jnp
import numpy as np
from jax.experimental import pallas as pl

_NB = 8
_SP = 24
_N = 2048
_TM = 256


def _knn_tile_kernel(gc_ref, pct_ref, val_ref, out_ref):
    gc = gc_ref[0]
    pct = pct_ref[0]
    val = val_ref[0]
    gn = jnp.sum(gc * gc, axis=1, keepdims=True)
    pn = jnp.sum(pct * pct, axis=0, keepdims=True)
    dot = jax.lax.dot_general(
        gc, pct, (((1,), (0,)), ((), ())),
        precision=jax.lax.Precision.DEFAULT,
        preferred_element_type=jnp.float32)
    work = pn - 2.0 * dot
    wsel = jnp.zeros_like(work)
    denom = jnp.zeros_like(gn)
    for _ in range(_NB):
        m = jnp.min(work, axis=1, keepdims=True)
        sel = work == m
        wi = 1.0 / jnp.square(jnp.sqrt(jnp.maximum(m + gn, 0.0)) + 1e-8)
        denom = denom + wi
        wsel = jnp.where(sel, wi, wsel)
        work = jnp.where(sel, jnp.float32(jnp.inf), work)
    out = jax.lax.dot_general(
        wsel, val, (((1,), (0,)), ((), ())),
        precision=jax.lax.Precision.HIGHEST,
        preferred_element_type=jnp.float32)
    out_ref[0] = out / denom


def _div_kernel(fx_ref, fy_ref, fz_ref, out_ref):
    h = 2.0 * np.pi / _SP
    fx = fx_ref[...]
    fy = fy_ref[...]
    fz = fz_ref[...]
    row = jax.lax.broadcasted_iota(jnp.int32, fx.shape, 0)
    y = row % _SP
    x = (row // _SP) % _SP
    upx = jnp.concatenate([fx[_SP:], fx[-_SP:]], axis=0)
    dnx = jnp.concatenate([fx[:_SP], fx[:-_SP]], axis=0)
    gx = (upx - dnx) / (2.0 * h)
    gx = jnp.where(x == 0, (upx - fx) / h, gx)
    gx = jnp.where(x == _SP - 1, (fx - dnx) / h, gx)
    upy = jnp.concatenate([fy[1:], fy[-1:]], axis=0)
    dny = jnp.concatenate([fy[:1], fy[:-1]], axis=0)
    gy = (upy - dny) / (2.0 * h)
    gy = jnp.where(y == 0, (upy - fy) / h, gy)
    gy = jnp.where(y == _SP - 1, (fy - dny) / h, gy)
    zc = (fz[:, 2:] - fz[:, :-2]) / (2.0 * h)
    z0 = (fz[:, 1:2] - fz[:, 0:1]) / h
    z1 = (fz[:, -1:] - fz[:, -2:-1]) / h
    gz = jnp.concatenate([z0, zc, z1], axis=1)
    div = gx + gy + gz
    out_ref[...] = jnp.broadcast_to(jnp.mean(jnp.abs(div)), (1, 1))


def kernel(flow, coords, grid_coords):
    B, N, _ = coords.shape
    M = grid_coords.shape[1]
    coords_t = jnp.transpose(coords, (0, 2, 1))
    nt = M // _TM
    gf = pl.pallas_call(
        _knn_tile_kernel,
        grid=(B, nt),
        in_specs=[
            pl.BlockSpec((1, _TM, 3), lambda b, i: (b, i, 0)),
            pl.BlockSpec((1, 3, N), lambda b, i: (b, 0, 0)),
            pl.BlockSpec((1, N, 3), lambda b, i: (b, 0, 0)),
        ],
        out_specs=pl.BlockSpec((1, _TM, 3), lambda b, i: (b, i, 0)),
        out_shape=jax.ShapeDtypeStruct((B, M, 3), jnp.float32),
    )(grid_coords, coords_t, flow)
    g = gf.reshape(B, _SP, _SP, _SP, 3)
    fx = g[..., 0].reshape(B * _SP * _SP, _SP)
    fy = g[..., 1].reshape(B * _SP * _SP, _SP)
    fz = g[..., 2].reshape(B * _SP * _SP, _SP)
    out = pl.pallas_call(
        _div_kernel,
        out_shape=jax.ShapeDtypeStruct((1, 1), jnp.float32),
    )(fx, fy, fz)
    return out[0, 0]


# speedup vs baseline: 19.6215x; 1.0295x over previous
import jax
import jax.numpy as jnp
import numpy as np
from jax.experimental import pallas as pl

_NB = 8
_SP = 24
_N = 2048
_TM = 512


def _knn_tile_kernel(gc_ref, pct_ref, val_ref, out_ref):
    gc = gc_ref[0]
    pct = pct_ref[0]
    val = val_ref[0]
    gn = jnp.sum(gc * gc, axis=1, keepdims=True)
    pn = jnp.sum(pct * pct, axis=0, keepdims=True)
    dot = jax.lax.dot_general(
        gc, pct, (((1,), (0,)), ((), ())),
        precision=jax.lax.Precision.DEFAULT,
        preferred_element_type=jnp.float32)
    work = pn - 2.0 * dot
    wsel = jnp.zeros_like(work)
    denom = jnp.zeros_like(gn)
    for _ in range(_NB):
        m = jnp.min(work, axis=1, keepdims=True)
        sel = work == m
        wi = 1.0 / jnp.square(jnp.sqrt(jnp.maximum(m + gn, 0.0)) + 1e-8)
        denom = denom + wi
        wsel = jnp.where(sel, wi, wsel)
        work = jnp.where(sel, jnp.float32(jnp.inf), work)
    out = jax.lax.dot_general(
        wsel, val, (((1,), (0,)), ((), ())),
        precision=jax.lax.Precision.HIGHEST,
        preferred_element_type=jnp.float32)
    out_ref[0] = out / denom


def _div_kernel(fx_ref, fy_ref, fz_ref, out_ref):
    h = 2.0 * np.pi / _SP
    fx = fx_ref[...]
    fy = fy_ref[...]
    fz = fz_ref[...]
    row = jax.lax.broadcasted_iota(jnp.int32, fx.shape, 0)
    y = row % _SP
    x = (row // _SP) % _SP
    upx = jnp.concatenate([fx[_SP:], fx[-_SP:]], axis=0)
    dnx = jnp.concatenate([fx[:_SP], fx[:-_SP]], axis=0)
    gx = (upx - dnx) / (2.0 * h)
    gx = jnp.where(x == 0, (upx - fx) / h, gx)
    gx = jnp.where(x == _SP - 1, (fx - dnx) / h, gx)
    upy = jnp.concatenate([fy[1:], fy[-1:]], axis=0)
    dny = jnp.concatenate([fy[:1], fy[:-1]], axis=0)
    gy = (upy - dny) / (2.0 * h)
    gy = jnp.where(y == 0, (upy - fy) / h, gy)
    gy = jnp.where(y == _SP - 1, (fy - dny) / h, gy)
    zc = (fz[:, 2:] - fz[:, :-2]) / (2.0 * h)
    z0 = (fz[:, 1:2] - fz[:, 0:1]) / h
    z1 = (fz[:, -1:] - fz[:, -2:-1]) / h
    gz = jnp.concatenate([z0, zc, z1], axis=1)
    div = gx + gy + gz
    out_ref[...] = jnp.broadcast_to(jnp.mean(jnp.abs(div)), (1, 1))


def kernel(flow, coords, grid_coords):
    B, N, _ = coords.shape
    M = grid_coords.shape[1]
    coords_t = jnp.transpose(coords, (0, 2, 1))
    nt = M // _TM
    gf = pl.pallas_call(
        _knn_tile_kernel,
        grid=(B, nt),
        in_specs=[
            pl.BlockSpec((1, _TM, 3), lambda b, i: (b, i, 0)),
            pl.BlockSpec((1, 3, N), lambda b, i: (b, 0, 0)),
            pl.BlockSpec((1, N, 3), lambda b, i: (b, 0, 0)),
        ],
        out_specs=pl.BlockSpec((1, _TM, 3), lambda b, i: (b, i, 0)),
        out_shape=jax.ShapeDtypeStruct((B, M, 3), jnp.float32),
    )(grid_coords, coords_t, flow)
    g = gf.reshape(B, _SP, _SP, _SP, 3)
    fx = g[..., 0].reshape(B * _SP * _SP, _SP)
    fy = g[..., 1].reshape(B * _SP * _SP, _SP)
    fz = g[..., 2].reshape(B * _SP * _SP, _SP)
    out = pl.pallas_call(
        _div_kernel,
        out_shape=jax.ShapeDtypeStruct((1, 1), jnp.float32),
    )(fx, fy, fz)
    return out[0, 0]


# speedup vs baseline: 25.9500x; 1.3225x over previous
import jax
import jax.numpy as jnp
import numpy as np
from jax.experimental import pallas as pl

_NB = 8
_SP = 24
_N = 2048
_TM = 512


def _knn_tile_kernel(gc_ref, pct_ref, val_ref, out_ref):
    gc = gc_ref[0]
    pct = pct_ref[0]
    val = val_ref[0]
    gn = jnp.sum(gc * gc, axis=1, keepdims=True)
    pn = jnp.sum(pct * pct, axis=0, keepdims=True)
    dot = jax.lax.dot_general(
        gc, pct, (((1,), (0,)), ((), ())),
        precision=jax.lax.Precision.DEFAULT,
        preferred_element_type=jnp.float32)
    work = pn - 2.0 * dot
    wsel = jnp.zeros_like(work)
    denom = jnp.zeros_like(gn)
    for _ in range(_NB):
        m = jnp.min(work, axis=1, keepdims=True)
        sel = work == m
        wi = 1.0 / jnp.square(jnp.sqrt(jnp.maximum(m + gn, 0.0)) + 1e-8)
        denom = denom + wi
        wsel = jnp.where(sel, wi, wsel)
        work = jnp.where(sel, jnp.float32(jnp.inf), work)
    out = jax.lax.dot_general(
        wsel, val, (((1,), (0,)), ((), ())),
        precision=jax.lax.Precision.DEFAULT,
        preferred_element_type=jnp.float32)
    out_ref[0] = out / denom


def _div_kernel(fx_ref, fy_ref, fz_ref, out_ref):
    h = 2.0 * np.pi / _SP
    fx = fx_ref[...]
    fy = fy_ref[...]
    fz = fz_ref[...]
    row = jax.lax.broadcasted_iota(jnp.int32, fx.shape, 0)
    y = row % _SP
    x = (row // _SP) % _SP
    upx = jnp.concatenate([fx[_SP:], fx[-_SP:]], axis=0)
    dnx = jnp.concatenate([fx[:_SP], fx[:-_SP]], axis=0)
    gx = (upx - dnx) / (2.0 * h)
    gx = jnp.where(x == 0, (upx - fx) / h, gx)
    gx = jnp.where(x == _SP - 1, (fx - dnx) / h, gx)
    upy = jnp.concatenate([fy[1:], fy[-1:]], axis=0)
    dny = jnp.concatenate([fy[:1], fy[:-1]], axis=0)
    gy = (upy - dny) / (2.0 * h)
    gy = jnp.where(y == 0, (upy - fy) / h, gy)
    gy = jnp.where(y == _SP - 1, (fy - dny) / h, gy)
    zc = (fz[:, 2:] - fz[:, :-2]) / (2.0 * h)
    z0 = (fz[:, 1:2] - fz[:, 0:1]) / h
    z1 = (fz[:, -1:] - fz[:, -2:-1]) / h
    gz = jnp.concatenate([z0, zc, z1], axis=1)
    div = gx + gy + gz
    out_ref[...] = jnp.broadcast_to(jnp.mean(jnp.abs(div)), (1, 1))


def kernel(flow, coords, grid_coords):
    B, N, _ = coords.shape
    M = grid_coords.shape[1]
    coords_t = jnp.transpose(coords, (0, 2, 1))
    nt = M // _TM
    gf = pl.pallas_call(
        _knn_tile_kernel,
        grid=(B, nt),
        in_specs=[
            pl.BlockSpec((1, _TM, 3), lambda b, i: (b, i, 0)),
            pl.BlockSpec((1, 3, N), lambda b, i: (b, 0, 0)),
            pl.BlockSpec((1, N, 3), lambda b, i: (b, 0, 0)),
        ],
        out_specs=pl.BlockSpec((1, _TM, 3), lambda b, i: (b, i, 0)),
        out_shape=jax.ShapeDtypeStruct((B, M, 3), jnp.float32),
    )(grid_coords, coords_t, flow)
    g = gf.reshape(B, _SP, _SP, _SP, 3)
    fx = g[..., 0].reshape(B * _SP * _SP, _SP)
    fy = g[..., 1].reshape(B * _SP * _SP, _SP)
    fz = g[..., 2].reshape(B * _SP * _SP, _SP)
    out = pl.pallas_call(
        _div_kernel,
        out_shape=jax.ShapeDtypeStruct((1, 1), jnp.float32),
    )(fx, fy, fz)
    return out[0, 0]


# speedup vs baseline: 26.0784x; 1.0049x over previous
import jax
import jax.numpy as jnp
import numpy as np
from jax.experimental import pallas as pl

_NB = 8
_SP = 24
_N = 2048
_TM = 768


def _knn_tile_kernel(gc_ref, pct_ref, val_ref, out_ref):
    gc = gc_ref[0]
    pct = pct_ref[0]
    val = val_ref[0]
    gn = jnp.sum(gc * gc, axis=1, keepdims=True)
    pn = jnp.sum(pct * pct, axis=0, keepdims=True)
    dot = jax.lax.dot_general(
        gc, pct, (((1,), (0,)), ((), ())),
        precision=jax.lax.Precision.DEFAULT,
        preferred_element_type=jnp.float32)
    work = pn - 2.0 * dot
    wsel = jnp.zeros_like(work)
    denom = jnp.zeros_like(gn)
    for _ in range(_NB):
        m = jnp.min(work, axis=1, keepdims=True)
        sel = work == m
        wi = 1.0 / jnp.square(jnp.sqrt(jnp.maximum(m + gn, 0.0)) + 1e-8)
        denom = denom + wi
        wsel = jnp.where(sel, wi, wsel)
        work = jnp.where(sel, jnp.float32(jnp.inf), work)
    out = jax.lax.dot_general(
        wsel, val, (((1,), (0,)), ((), ())),
        precision=jax.lax.Precision.DEFAULT,
        preferred_element_type=jnp.float32)
    out_ref[0] = out / denom


def _div_kernel(fx_ref, fy_ref, fz_ref, out_ref):
    h = 2.0 * np.pi / _SP
    fx = fx_ref[...]
    fy = fy_ref[...]
    fz = fz_ref[...]
    row = jax.lax.broadcasted_iota(jnp.int32, fx.shape, 0)
    y = row % _SP
    x = (row // _SP) % _SP
    upx = jnp.concatenate([fx[_SP:], fx[-_SP:]], axis=0)
    dnx = jnp.concatenate([fx[:_SP], fx[:-_SP]], axis=0)
    gx = (upx - dnx) / (2.0 * h)
    gx = jnp.where(x == 0, (upx - fx) / h, gx)
    gx = jnp.where(x == _SP - 1, (fx - dnx) / h, gx)
    upy = jnp.concatenate([fy[1:], fy[-1:]], axis=0)
    dny = jnp.concatenate([fy[:1], fy[:-1]], axis=0)
    gy = (upy - dny) / (2.0 * h)
    gy = jnp.where(y == 0, (upy - fy) / h, gy)
    gy = jnp.where(y == _SP - 1, (fy - dny) / h, gy)
    zc = (fz[:, 2:] - fz[:, :-2]) / (2.0 * h)
    z0 = (fz[:, 1:2] - fz[:, 0:1]) / h
    z1 = (fz[:, -1:] - fz[:, -2:-1]) / h
    gz = jnp.concatenate([z0, zc, z1], axis=1)
    div = gx + gy + gz
    out_ref[...] = jnp.broadcast_to(jnp.mean(jnp.abs(div)), (1, 1))


def kernel(flow, coords, grid_coords):
    B, N, _ = coords.shape
    M = grid_coords.shape[1]
    coords_t = jnp.transpose(coords, (0, 2, 1))
    nt = M // _TM
    gf = pl.pallas_call(
        _knn_tile_kernel,
        grid=(B, nt),
        in_specs=[
            pl.BlockSpec((1, _TM, 3), lambda b, i: (b, i, 0)),
            pl.BlockSpec((1, 3, N), lambda b, i: (b, 0, 0)),
            pl.BlockSpec((1, N, 3), lambda b, i: (b, 0, 0)),
        ],
        out_specs=pl.BlockSpec((1, _TM, 3), lambda b, i: (b, i, 0)),
        out_shape=jax.ShapeDtypeStruct((B, M, 3), jnp.float32),
    )(grid_coords, coords_t, flow)
    g = gf.reshape(B, _SP, _SP, _SP, 3)
    fx = g[..., 0].reshape(B * _SP * _SP, _SP)
    fy = g[..., 1].reshape(B * _SP * _SP, _SP)
    fz = g[..., 2].reshape(B * _SP * _SP, _SP)
    out = pl.pallas_call(
        _div_kernel,
        out_shape=jax.ShapeDtypeStruct((1, 1), jnp.float32),
    )(fx, fy, fz)
    return out[0, 0]
